# baseline (speedup 1.0000x reference)
import functools

import jax
import jax.numpy as jnp
from jax import lax
from jax.experimental import pallas as pl
from jax.experimental.pallas import tpu as pltpu
from jax.experimental.pallas import tpu_sc as plsc

N = 10000
NP = 10240
E = 320000
DF = 128
DH = 32
DT = 48
NC = 2
NS = 16
NW = NC * NS
CH = 80
EPW = E // NW
EPWP = 10240
PAD = EPWP - EPW
RPW = EPWP // CH
NROWS = NW * RPW
NB = 4
LEAD = 2
NPT = NP // NS
SUB = 320

_MESH = dict(core_axis_name="c", subcore_axis_name="s", num_cores=NC,
             num_subcores=NS)
_SC_PARAMS = pltpu.CompilerParams(use_tc_tiling_on_sc=False,
                                  needs_layout_passes=False)


def _pipelined_agg(table, src_v, dst_v, ring_v, acc_sh, gsems, ssems):
    def _gather(j, b):
        return pltpu.async_copy(table.at[src_v.at[j]], ring_v.at[b], gsems[b])

    for b in range(LEAD):
        _gather(b, b)

    def outer(t, carry):
        for b in range(NB):
            j = t * NB + b
            pltpu.make_async_copy(table.at[src_v.at[j]], ring_v.at[b],
                                  gsems[b]).wait()
            pltpu.async_copy(ring_v.at[b], acc_sh.at[dst_v.at[j]],
                             ssems[b], add=True)
            bn = (b + LEAD) % NB
            jn = j + LEAD

            @pl.when(jnp.logical_and(jn < RPW, j >= NB - LEAD))
            def _():
                pltpu.make_async_copy(ring_v.at[bn], acc_sh.at[dst_v.at[0]],
                                      ssems[bn]).wait()

            @pl.when(jn < RPW)
            def _():
                _gather(jn, bn)
        return carry

    lax.fori_loop(0, RPW // NB, outer, 0)
    for b in range(NB):
        pltpu.make_async_copy(ring_v.at[b], acc_sh.at[dst_v.at[0]],
                              ssems[b]).wait()


def _sc_mega(y1t, src2, dst2, z48, z1, r1, wpack,
             out_hbm, acc1p, acc2p,
             src_v, dst_v, rows_v, vals_v,
             accA_v, accB_v, r1_v, y2loc, r2loc, invloc,
             a2a_v, a2b_v, p4inv_v, p4r2_v, out_v, wpack_v,
             acc1_sh, acc2_sh, y2_sh, r2_sh, invd_sh,
             g0, g1, g2, g3, s0, s1, s2, s3, csem):
    gsems = [g0, g1, g2, g3]
    ssems = [s0, s1, s2, s3]
    c = lax.axis_index("c")
    s = lax.axis_index("s")
    oc = 1 - c
    wid = c * NS + s
    nbase = s * NPT
    iota = lax.iota(jnp.int32, 16)

    pltpu.sync_copy(z48.at[pl.ds(nbase, NPT)], acc1_sh.at[pl.ds(nbase, NPT)])
    pltpu.sync_copy(z1.at[pl.ds(nbase, NPT)], acc2_sh.at[pl.ds(nbase, NPT)])
    pltpu.sync_copy(src2.at[pl.ds(wid * RPW, RPW)], src_v)
    pltpu.sync_copy(dst2.at[pl.ds(wid * RPW, RPW)], dst_v)
    pltpu.sync_copy(wpack, wpack_v)
    plsc.subcore_barrier()

    _pipelined_agg(y1t, src_v, dst_v, rows_v, acc1_sh, gsems, ssems)
    plsc.subcore_barrier()

    @pl.when(s == 0)
    def _():
        pltpu.sync_copy(acc1_sh, acc1p.at[c])
        pltpu.core_barrier(csem, core_axis_name="c")
    plsc.subcore_barrier()

    for sub in range(NPT // SUB):
        off = nbase + sub * SUB
        pltpu.sync_copy(acc1_sh.at[pl.ds(off, SUB)], accA_v)
        pltpu.sync_copy(acc1p.at[oc, pl.ds(off, SUB)], accB_v)
        pltpu.sync_copy(r1.at[pl.ds(off, SUB)], r1_v)

        def group(g, carry):
            base = g * 16
            rows = base + iota
            degc = jnp.full((16,), DH, jnp.int32)
            deg = (plsc.load_gather(accA_v, [rows, degc])
                   + plsc.load_gather(accB_v, [rows, degc]))
            inv = 1.0 / jnp.maximum(deg, 1.0)
            y2a = jnp.zeros((16,), jnp.float32)
            r2a = wpack_v[2 * DH, :]
            for col in range(DH):
                colv = jnp.full((16,), col, jnp.int32)
                a = (plsc.load_gather(accA_v, [rows, colv])
                     + plsc.load_gather(accB_v, [rows, colv]))
                r1c = plsc.load_gather(r1_v, [rows, colv])
                hc = jnp.maximum(a * inv + r1c, 0.0)
                y2a = y2a + hc * wpack_v[col, :]
                r2a = r2a + hc * wpack_v[DH + col, :]
            y2loc[pl.ds(base, 16)] = y2a
            r2loc[pl.ds(base, 16)] = r2a
            invloc[pl.ds(base, 16)] = inv
            return carry

        lax.fori_loop(0, SUB // 16, group, 0)
        pltpu.sync_copy(y2loc, y2_sh.at[pl.ds(off, SUB)])
        pltpu.sync_copy(r2loc, r2_sh.at[pl.ds(off, SUB)])
        pltpu.sync_copy(invloc, invd_sh.at[pl.ds(off, SUB)])
    plsc.subcore_barrier()

    _pipelined_agg(y2_sh, src_v, dst_v, vals_v, acc2_sh, gsems, ssems)
    plsc.subcore_barrier()

    @pl.when(s == 0)
    def _():
        pltpu.sync_copy(acc2_sh, acc2p.at[c])
        pltpu.core_barrier(csem, core_axis_name="c")
    plsc.subcore_barrier()

    @pl.when(c == 0)
    def _():
        pltpu.sync_copy(acc2_sh.at[pl.ds(nbase, NPT)], a2a_v)
        pltpu.sync_copy(acc2p.at[1, pl.ds(nbase, NPT)], a2b_v)
        pltpu.sync_copy(invd_sh.at[pl.ds(nbase, NPT)], p4inv_v)
        pltpu.sync_copy(r2_sh.at[pl.ds(nbase, NPT)], p4r2_v)
        for i in range(NPT // 16):
            d = pl.ds(i * 16, 16)
            out_v[d] = (a2a_v[d] + a2b_v[d]) * p4inv_v[d] + p4r2_v[d]
        pltpu.sync_copy(out_v, out_hbm.at[pl.ds(nbase, NPT)])


def _tc_pre(x_ref, wlt_ref, wrt_ref, ones_row_ref, b1_row_ref,
            y1t_ref, r1_ref):
    xv = x_ref[...]
    y1t_ref[0:N, :] = (jnp.dot(xv, wlt_ref[...],
                               preferred_element_type=jnp.float32)
                       + ones_row_ref[...])
    y1t_ref[N:NP, :] = jnp.zeros((NP - N, DT), jnp.float32)
    r1_ref[0:N, :] = (jnp.dot(xv, wrt_ref[...],
                              preferred_element_type=jnp.float32)
                      + b1_row_ref[...])
    r1_ref[N:NP, :] = jnp.zeros((NP - N, DH), jnp.float32)


def kernel(x, edge_index, W1_l, b1, W1_r, W2_l, b2, W2_r):
    ei = edge_index.astype(jnp.int32)
    srcp = jnp.pad(ei[0].reshape(NW, EPW), ((0, 0), (0, PAD)),
                   constant_values=N)
    dstp = jnp.pad(ei[1].reshape(NW, EPW), ((0, 0), (0, PAD)),
                   constant_values=NP - 1)
    src2 = srcp.reshape(NROWS, CH)
    dst2 = dstp.reshape(NROWS, CH)
    z48 = jnp.zeros((NP, DT), jnp.float32)
    z1 = jnp.zeros((NP,), jnp.float32)
    wlt = jnp.zeros((DF, DT), jnp.float32).at[:, :DH].set(W1_l.T)
    ones_row = jnp.zeros((1, DT), jnp.float32).at[0, DH].set(1.0)
    wpack = jnp.tile(jnp.concatenate([W2_l.reshape(DH), W2_r.reshape(DH),
                                      b2])[:, None], (1, 16))

    y1t, r1 = pl.pallas_call(
        _tc_pre,
        out_shape=[jax.ShapeDtypeStruct((NP, DT), jnp.float32),
                   jax.ShapeDtypeStruct((NP, DH), jnp.float32)],
    )(x, wlt, W1_r.T, ones_row, b1.reshape(1, DH))

    scratch = [
        pltpu.VMEM((RPW, CH), jnp.int32),
        pltpu.VMEM((RPW, CH), jnp.int32),
        pltpu.VMEM((NB, CH, DT), jnp.float32),
        pltpu.VMEM((NB, CH), jnp.float32),
        pltpu.VMEM((SUB, DT), jnp.float32),
        pltpu.VMEM((SUB, DT), jnp.float32),
        pltpu.VMEM((SUB, DH), jnp.float32),
        pltpu.VMEM((SUB,), jnp.float32),
        pltpu.VMEM((SUB,), jnp.float32),
        pltpu.VMEM((SUB,), jnp.float32),
        pltpu.VMEM((NPT,), jnp.float32),
        pltpu.VMEM((NPT,), jnp.float32),
        pltpu.VMEM((NPT,), jnp.float32),
        pltpu.VMEM((NPT,), jnp.float32),
        pltpu.VMEM((NPT,), jnp.float32),
        pltpu.VMEM((2 * DH + 1, 16), jnp.float32),
        pltpu.VMEM_SHARED((NP, DT), jnp.float32),
        pltpu.VMEM_SHARED((NP,), jnp.float32),
        pltpu.VMEM_SHARED((NP,), jnp.float32),
        pltpu.VMEM_SHARED((NP,), jnp.float32),
        pltpu.VMEM_SHARED((NP,), jnp.float32),
    ] + [pltpu.SemaphoreType.DMA] * (2 * NB) + [pltpu.SemaphoreType.REGULAR]
    out_p, _, _ = pl.kernel(
        _sc_mega,
        out_type=[jax.ShapeDtypeStruct((NP,), jnp.float32),
                  jax.ShapeDtypeStruct((NC, NP, DT), jnp.float32),
                  jax.ShapeDtypeStruct((NC, NP), jnp.float32)],
        mesh=plsc.VectorSubcoreMesh(**_MESH),
        scratch_types=scratch,
        compiler_params=_SC_PARAMS,
    )(y1t, src2, dst2, z48, z1, r1, wpack)
    return out_p[:N].reshape(N, 1)


# speedup vs baseline: 1.1111x; 1.1111x over previous
import functools

import jax
import jax.numpy as jnp
from jax import lax
from jax.experimental import pallas as pl
from jax.experimental.pallas import tpu as pltpu
from jax.experimental.pallas import tpu_sc as plsc

N = 10000
NP = 10240
E = 320000
DF = 128
DH = 32
DT = 40
NC = 2
NS = 16
NW = NC * NS
CH = 80
EPW = E // NW
EPWP = 10240
PAD = EPWP - EPW
RPW = EPWP // CH
NROWS = NW * RPW
NB = 8
LEAD = 4
NPT = NP // NS
SUB = 320

_MESH = dict(core_axis_name="c", subcore_axis_name="s", num_cores=NC,
             num_subcores=NS)
_SC_PARAMS = pltpu.CompilerParams(use_tc_tiling_on_sc=False,
                                  needs_layout_passes=False)


def _pipelined_agg(table, src_v, dst_v, ring_v, acc_sh, gsems, ssems):
    def _gather(j, b):
        return pltpu.async_copy(table.at[src_v.at[j]], ring_v.at[b], gsems[b])

    for b in range(LEAD):
        _gather(b, b)

    def outer(t, carry):
        for b in range(NB):
            j = t * NB + b
            pltpu.make_async_copy(table.at[src_v.at[j]], ring_v.at[b],
                                  gsems[b]).wait()
            pltpu.async_copy(ring_v.at[b], acc_sh.at[dst_v.at[j]],
                             ssems[b], add=True)
            bn = (b + LEAD) % NB
            jn = j + LEAD

            @pl.when(jnp.logical_and(jn < RPW, j >= NB - LEAD))
            def _():
                pltpu.make_async_copy(ring_v.at[bn], acc_sh.at[dst_v.at[0]],
                                      ssems[bn]).wait()

            @pl.when(jn < RPW)
            def _():
                _gather(jn, bn)
        return carry

    lax.fori_loop(0, RPW // NB, outer, 0)
    for b in range(NB):
        pltpu.make_async_copy(ring_v.at[b], acc_sh.at[dst_v.at[0]],
                              ssems[b]).wait()


def _sc_mega(y1t, src2, dst2, z48, z1, r1, wpack,
             out_hbm, acc1p, acc2p,
             src_v, dst_v, rows_v, vals_v,
             accA_v, accB_v, r1_v, y2loc, r2loc, invloc,
             a2a_v, a2b_v, p4inv_v, p4r2_v, out_v, wpack_v,
             acc1_sh, acc2_sh, y2_sh, r2_sh, invd_sh,
             g0, g1, g2, g3, g4, g5, g6, g7,
             s0, s1, s2, s3, s4, s5, s6, s7, csem):
    gsems = [g0, g1, g2, g3, g4, g5, g6, g7]
    ssems = [s0, s1, s2, s3, s4, s5, s6, s7]
    c = lax.axis_index("c")
    s = lax.axis_index("s")
    oc = 1 - c
    wid = c * NS + s
    nbase = s * NPT
    iota = lax.iota(jnp.int32, 16)

    pltpu.sync_copy(z48.at[pl.ds(nbase, NPT)], acc1_sh.at[pl.ds(nbase, NPT)])
    pltpu.sync_copy(z1.at[pl.ds(nbase, NPT)], acc2_sh.at[pl.ds(nbase, NPT)])
    pltpu.sync_copy(src2.at[pl.ds(wid * RPW, RPW)], src_v)
    pltpu.sync_copy(dst2.at[pl.ds(wid * RPW, RPW)], dst_v)
    pltpu.sync_copy(wpack, wpack_v)
    plsc.subcore_barrier()

    _pipelined_agg(y1t, src_v, dst_v, rows_v, acc1_sh, gsems, ssems)
    plsc.subcore_barrier()

    @pl.when(s == 0)
    def _():
        pltpu.sync_copy(acc1_sh, acc1p.at[c])
        pltpu.core_barrier(csem, core_axis_name="c")
    plsc.subcore_barrier()

    for sub in range(NPT // SUB):
        off = nbase + sub * SUB
        pltpu.sync_copy(acc1_sh.at[pl.ds(off, SUB)], accA_v)
        pltpu.sync_copy(acc1p.at[oc, pl.ds(off, SUB)], accB_v)
        pltpu.sync_copy(r1.at[pl.ds(off, SUB)], r1_v)

        def group(g, carry):
            base = g * 16
            rows = base + iota
            degc = jnp.full((16,), DH, jnp.int32)
            deg = (plsc.load_gather(accA_v, [rows, degc])
                   + plsc.load_gather(accB_v, [rows, degc]))
            inv = 1.0 / jnp.maximum(deg, 1.0)
            y2a = jnp.zeros((16,), jnp.float32)
            r2a = wpack_v[2 * DH, :]
            for col in range(DH):
                colv = jnp.full((16,), col, jnp.int32)
                a = (plsc.load_gather(accA_v, [rows, colv])
                     + plsc.load_gather(accB_v, [rows, colv]))
                r1c = plsc.load_gather(r1_v, [rows, colv])
                hc = jnp.maximum(a * inv + r1c, 0.0)
                y2a = y2a + hc * wpack_v[col, :]
                r2a = r2a + hc * wpack_v[DH + col, :]
            y2loc[pl.ds(base, 16)] = y2a
            r2loc[pl.ds(base, 16)] = r2a
            invloc[pl.ds(base, 16)] = inv
            return carry

        lax.fori_loop(0, SUB // 16, group, 0)
        pltpu.sync_copy(y2loc, y2_sh.at[pl.ds(off, SUB)])
        pltpu.sync_copy(r2loc, r2_sh.at[pl.ds(off, SUB)])
        pltpu.sync_copy(invloc, invd_sh.at[pl.ds(off, SUB)])
    plsc.subcore_barrier()

    _pipelined_agg(y2_sh, src_v, dst_v, vals_v, acc2_sh, gsems, ssems)
    plsc.subcore_barrier()

    @pl.when(s == 0)
    def _():
        pltpu.sync_copy(acc2_sh, acc2p.at[c])
        pltpu.core_barrier(csem, core_axis_name="c")
    plsc.subcore_barrier()

    @pl.when(c == 0)
    def _():
        pltpu.sync_copy(acc2_sh.at[pl.ds(nbase, NPT)], a2a_v)
        pltpu.sync_copy(acc2p.at[1, pl.ds(nbase, NPT)], a2b_v)
        pltpu.sync_copy(invd_sh.at[pl.ds(nbase, NPT)], p4inv_v)
        pltpu.sync_copy(r2_sh.at[pl.ds(nbase, NPT)], p4r2_v)
        for i in range(NPT // 16):
            d = pl.ds(i * 16, 16)
            out_v[d] = (a2a_v[d] + a2b_v[d]) * p4inv_v[d] + p4r2_v[d]
        pltpu.sync_copy(out_v, out_hbm.at[pl.ds(nbase, NPT)])


def _tc_pre(x_ref, wlt_ref, wrt_ref, ones_row_ref, b1_row_ref,
            y1t_ref, r1_ref):
    xv = x_ref[...]
    y1t_ref[0:N, :] = (jnp.dot(xv, wlt_ref[...],
                               preferred_element_type=jnp.float32)
                       + ones_row_ref[...])
    y1t_ref[N:NP, :] = jnp.zeros((NP - N, DT), jnp.float32)
    r1_ref[0:N, :] = (jnp.dot(xv, wrt_ref[...],
                              preferred_element_type=jnp.float32)
                      + b1_row_ref[...])
    r1_ref[N:NP, :] = jnp.zeros((NP - N, DH), jnp.float32)


def kernel(x, edge_index, W1_l, b1, W1_r, W2_l, b2, W2_r):
    ei = edge_index.astype(jnp.int32)
    srcp = jnp.pad(ei[0].reshape(NW, EPW), ((0, 0), (0, PAD)),
                   constant_values=N)
    dstp = jnp.pad(ei[1].reshape(NW, EPW), ((0, 0), (0, PAD)),
                   constant_values=NP - 1)
    src2 = srcp.reshape(NROWS, CH)
    dst2 = dstp.reshape(NROWS, CH)
    z48 = jnp.zeros((NP, DT), jnp.float32)
    z1 = jnp.zeros((NP,), jnp.float32)
    wlt = jnp.zeros((DF, DT), jnp.float32).at[:, :DH].set(W1_l.T)
    ones_row = jnp.zeros((1, DT), jnp.float32).at[0, DH].set(1.0)
    wpack = jnp.tile(jnp.concatenate([W2_l.reshape(DH), W2_r.reshape(DH),
                                      b2])[:, None], (1, 16))

    y1t, r1 = pl.pallas_call(
        _tc_pre,
        out_shape=[jax.ShapeDtypeStruct((NP, DT), jnp.float32),
                   jax.ShapeDtypeStruct((NP, DH), jnp.float32)],
    )(x, wlt, W1_r.T, ones_row, b1.reshape(1, DH))

    scratch = [
        pltpu.VMEM((RPW, CH), jnp.int32),
        pltpu.VMEM((RPW, CH), jnp.int32),
        pltpu.VMEM((NB, CH, DT), jnp.float32),
        pltpu.VMEM((NB, CH), jnp.float32),
        pltpu.VMEM((SUB, DT), jnp.float32),
        pltpu.VMEM((SUB, DT), jnp.float32),
        pltpu.VMEM((SUB, DH), jnp.float32),
        pltpu.VMEM((SUB,), jnp.float32),
        pltpu.VMEM((SUB,), jnp.float32),
        pltpu.VMEM((SUB,), jnp.float32),
        pltpu.VMEM((NPT,), jnp.float32),
        pltpu.VMEM((NPT,), jnp.float32),
        pltpu.VMEM((NPT,), jnp.float32),
        pltpu.VMEM((NPT,), jnp.float32),
        pltpu.VMEM((NPT,), jnp.float32),
        pltpu.VMEM((2 * DH + 1, 16), jnp.float32),
        pltpu.VMEM_SHARED((NP, DT), jnp.float32),
        pltpu.VMEM_SHARED((NP,), jnp.float32),
        pltpu.VMEM_SHARED((NP,), jnp.float32),
        pltpu.VMEM_SHARED((NP,), jnp.float32),
        pltpu.VMEM_SHARED((NP,), jnp.float32),
    ] + [pltpu.SemaphoreType.DMA] * (2 * NB) + [pltpu.SemaphoreType.REGULAR]
    out_p, _, _ = pl.kernel(
        _sc_mega,
        out_type=[jax.ShapeDtypeStruct((NP,), jnp.float32),
                  jax.ShapeDtypeStruct((NC, NP, DT), jnp.float32),
                  jax.ShapeDtypeStruct((NC, NP), jnp.float32)],
        mesh=plsc.VectorSubcoreMesh(**_MESH),
        scratch_types=scratch,
        compiler_params=_SC_PARAMS,
    )(y1t, src2, dst2, z48, z1, r1, wpack)
    return out_p[:N].reshape(N, 1)


# speedup vs baseline: 1.1202x; 1.0082x over previous
import functools

import jax
import jax.numpy as jnp
from jax import lax
from jax.experimental import pallas as pl
from jax.experimental.pallas import tpu as pltpu
from jax.experimental.pallas import tpu_sc as plsc

N = 10000
NP = 10240
E = 320000
DF = 128
DH = 32
DT = 40
NC = 2
NS = 16
NW = NC * NS
CH = 80
EPW = E // NW
EPWP = 10240
PAD = EPWP - EPW
RPW = EPWP // CH
NROWS = NW * RPW
NB = 8
LEAD = 4
NPT = NP // NS
SUB = 320

_MESH = dict(core_axis_name="c", subcore_axis_name="s", num_cores=NC,
             num_subcores=NS)
_SC_PARAMS = pltpu.CompilerParams(use_tc_tiling_on_sc=False,
                                  needs_layout_passes=False)


def _pipelined_agg(table, src_v, dst_v, ring_v, acc_sh, gsems, ssems):
    def _gather(j, b):
        return pltpu.async_copy(table.at[src_v.at[j]], ring_v.at[b], gsems[b])

    for b in range(LEAD):
        _gather(b, b)

    def outer(t, carry):
        for b in range(NB):
            j = t * NB + b
            pltpu.make_async_copy(table.at[src_v.at[j]], ring_v.at[b],
                                  gsems[b]).wait()
            pltpu.async_copy(ring_v.at[b], acc_sh.at[dst_v.at[j]],
                             ssems[b], add=True)
            bn = (b + LEAD) % NB
            jn = j + LEAD

            @pl.when(jnp.logical_and(jn < RPW, j >= NB - LEAD))
            def _():
                pltpu.make_async_copy(ring_v.at[bn], acc_sh.at[dst_v.at[0]],
                                      ssems[bn]).wait()

            @pl.when(jn < RPW)
            def _():
                _gather(jn, bn)
        return carry

    lax.fori_loop(0, RPW // NB, outer, 0)
    for b in range(NB):
        pltpu.make_async_copy(ring_v.at[b], acc_sh.at[dst_v.at[0]],
                              ssems[b]).wait()


def _sc_mega(y1t, src2, dst2, z48, z1, r1, wpack,
             out_hbm, acc1p, acc2p,
             src_v, dst_v, rows_v, vals_v,
             accA_v, accB_v, r1_v, y2loc, r2loc, invloc,
             a2a_v, a2b_v, p4inv_v, p4r2_v, out_v, wpack_v,
             acc1_sh, acc2_sh, y2_sh, r2_sh, invd_sh,
             g0, g1, g2, g3, g4, g5, g6, g7,
             s0, s1, s2, s3, s4, s5, s6, s7, csem):
    gsems = [g0, g1, g2, g3, g4, g5, g6, g7]
    ssems = [s0, s1, s2, s3, s4, s5, s6, s7]
    c = lax.axis_index("c")
    s = lax.axis_index("s")
    oc = 1 - c
    wid = c * NS + s
    nbase = s * NPT
    iota = lax.iota(jnp.int32, 16)

    pltpu.sync_copy(z48.at[pl.ds(nbase, NPT)], acc1_sh.at[pl.ds(nbase, NPT)])
    pltpu.sync_copy(z1.at[pl.ds(nbase, NPT)], acc2_sh.at[pl.ds(nbase, NPT)])
    pltpu.sync_copy(src2.at[pl.ds(wid * RPW, RPW)], src_v)
    pltpu.sync_copy(dst2.at[pl.ds(wid * RPW, RPW)], dst_v)
    pltpu.sync_copy(wpack, wpack_v)
    plsc.subcore_barrier()

    _pipelined_agg(y1t, src_v, dst_v, rows_v, acc1_sh, gsems, ssems)
    plsc.subcore_barrier()

    @pl.when(s == 0)
    def _():
        pltpu.sync_copy(acc1_sh, acc1p.at[c])
        pltpu.core_barrier(csem, core_axis_name="c")
    plsc.subcore_barrier()

    for sub in range(NPT // SUB):
        off = nbase + sub * SUB
        pltpu.sync_copy(acc1_sh.at[pl.ds(off, SUB)], accA_v)
        pltpu.sync_copy(acc1p.at[oc, pl.ds(off, SUB)], accB_v)
        pltpu.sync_copy(r1.at[pl.ds(off, SUB)], r1_v)

        def group(g, carry):
            base = g * 16
            rows = base + iota
            degc = jnp.full((16,), DH, jnp.int32)
            deg = (plsc.load_gather(accA_v, [rows, degc])
                   + plsc.load_gather(accB_v, [rows, degc]))
            inv = 1.0 / jnp.maximum(deg, 1.0)
            y2a = jnp.zeros((16,), jnp.float32)
            r2a = wpack_v[2 * DH, :]
            for col in range(DH):
                colv = jnp.full((16,), col, jnp.int32)
                a = (plsc.load_gather(accA_v, [rows, colv])
                     + plsc.load_gather(accB_v, [rows, colv]))
                r1c = plsc.load_gather(r1_v, [rows, colv])
                hc = jnp.maximum(a * inv + r1c, 0.0)
                y2a = y2a + hc * wpack_v[col, :]
                r2a = r2a + hc * wpack_v[DH + col, :]
            y2loc[pl.ds(base, 16)] = y2a
            r2loc[pl.ds(base, 16)] = r2a
            invloc[pl.ds(base, 16)] = inv
            return carry

        lax.fori_loop(0, SUB // 16, group, 0)
        pltpu.sync_copy(y2loc, y2_sh.at[pl.ds(off, SUB)])
        pltpu.sync_copy(r2loc, r2_sh.at[pl.ds(off, SUB)])
        pltpu.sync_copy(invloc, invd_sh.at[pl.ds(off, SUB)])
    plsc.subcore_barrier()

    _pipelined_agg(y2_sh, src_v, dst_v, vals_v, acc2_sh, gsems, ssems)
    plsc.subcore_barrier()

    @pl.when(s == 0)
    def _():
        pltpu.sync_copy(acc2_sh, acc2p.at[c])
        pltpu.core_barrier(csem, core_axis_name="c")
    plsc.subcore_barrier()

    @pl.when(c == 0)
    def _():
        pltpu.sync_copy(acc2_sh.at[pl.ds(nbase, NPT)], a2a_v)
        pltpu.sync_copy(acc2p.at[1, pl.ds(nbase, NPT)], a2b_v)
        pltpu.sync_copy(invd_sh.at[pl.ds(nbase, NPT)], p4inv_v)
        pltpu.sync_copy(r2_sh.at[pl.ds(nbase, NPT)], p4r2_v)
        for i in range(NPT // 16):
            d = pl.ds(i * 16, 16)
            out_v[d] = (a2a_v[d] + a2b_v[d]) * p4inv_v[d] + p4r2_v[d]
        pltpu.sync_copy(out_v, out_hbm.at[pl.ds(nbase, NPT)])


def _tc_pre(x_ref, wlt_ref, wrt_ref, ones_row_ref, b1_row_ref,
            y1t_ref, r1_ref):
    xv = x_ref[...]
    y1t_ref[0:N, :] = (jnp.dot(xv, wlt_ref[...],
                               preferred_element_type=jnp.float32)
                       + ones_row_ref[...])
    y1t_ref[N:NP, :] = jnp.zeros((NP - N, DT), jnp.float32)
    r1_ref[0:N, :] = (jnp.dot(xv, wrt_ref[...],
                              preferred_element_type=jnp.float32)
                      + b1_row_ref[...])
    r1_ref[N:NP, :] = jnp.zeros((NP - N, DH), jnp.float32)


def kernel(x, edge_index, W1_l, b1, W1_r, W2_l, b2, W2_r):
    ei = edge_index.astype(jnp.int32)
    srcp = jnp.pad(ei[0].reshape(NW, EPW), ((0, 0), (0, PAD)),
                   constant_values=N)
    dstp = jnp.concatenate(
        [ei[1].reshape(NW, EPW),
         jnp.broadcast_to((N + jnp.arange(NW, dtype=jnp.int32))[:, None],
                          (NW, PAD))], axis=1)
    src2 = srcp.reshape(NROWS, CH)
    dst2 = dstp.reshape(NROWS, CH)
    z48 = jnp.zeros((NP, DT), jnp.float32)
    z1 = jnp.zeros((NP,), jnp.float32)
    wlt = jnp.zeros((DF, DT), jnp.float32).at[:, :DH].set(W1_l.T)
    ones_row = jnp.zeros((1, DT), jnp.float32).at[0, DH].set(1.0)
    wpack = jnp.tile(jnp.concatenate([W2_l.reshape(DH), W2_r.reshape(DH),
                                      b2])[:, None], (1, 16))

    y1t, r1 = pl.pallas_call(
        _tc_pre,
        out_shape=[jax.ShapeDtypeStruct((NP, DT), jnp.float32),
                   jax.ShapeDtypeStruct((NP, DH), jnp.float32)],
    )(x, wlt, W1_r.T, ones_row, b1.reshape(1, DH))

    scratch = [
        pltpu.VMEM((RPW, CH), jnp.int32),
        pltpu.VMEM((RPW, CH), jnp.int32),
        pltpu.VMEM((NB, CH, DT), jnp.float32),
        pltpu.VMEM((NB, CH), jnp.float32),
        pltpu.VMEM((SUB, DT), jnp.float32),
        pltpu.VMEM((SUB, DT), jnp.float32),
        pltpu.VMEM((SUB, DH), jnp.float32),
        pltpu.VMEM((SUB,), jnp.float32),
        pltpu.VMEM((SUB,), jnp.float32),
        pltpu.VMEM((SUB,), jnp.float32),
        pltpu.VMEM((NPT,), jnp.float32),
        pltpu.VMEM((NPT,), jnp.float32),
        pltpu.VMEM((NPT,), jnp.float32),
        pltpu.VMEM((NPT,), jnp.float32),
        pltpu.VMEM((NPT,), jnp.float32),
        pltpu.VMEM((2 * DH + 1, 16), jnp.float32),
        pltpu.VMEM_SHARED((NP, DT), jnp.float32),
        pltpu.VMEM_SHARED((NP,), jnp.float32),
        pltpu.VMEM_SHARED((NP,), jnp.float32),
        pltpu.VMEM_SHARED((NP,), jnp.float32),
        pltpu.VMEM_SHARED((NP,), jnp.float32),
    ] + [pltpu.SemaphoreType.DMA] * (2 * NB) + [pltpu.SemaphoreType.REGULAR]
    out_p, _, _ = pl.kernel(
        _sc_mega,
        out_type=[jax.ShapeDtypeStruct((NP,), jnp.float32),
                  jax.ShapeDtypeStruct((NC, NP, DT), jnp.float32),
                  jax.ShapeDtypeStruct((NC, NP), jnp.float32)],
        mesh=plsc.VectorSubcoreMesh(**_MESH),
        scratch_types=scratch,
        compiler_params=_SC_PARAMS,
    )(y1t, src2, dst2, z48, z1, r1, wpack)
    return out_p[:N].reshape(N, 1)
